# baseline (speedup 1.0000x reference)
import jax
import jax.numpy as jnp
from jax import lax
from jax.experimental import pallas as pl
from jax.experimental.pallas import tpu as pltpu

HIDDEN = 768
NUM_EXPERTS = 8
TOP_K = 2
DFF = 1024
TOKENS = 2048
EPS = 1e-06


def _moe_body(hid_ref, rin_ref, rscale_ref, rproj_ref, pes_ref,
              w1_ref, w3_ref, w2_ref, out_ref, comb_ref, xbf_ref):
    e = pl.program_id(0)

    @pl.when(e == 0)
    def _():
        xbf_ref[...] = hid_ref[...].astype(jnp.bfloat16)
        x = rin_ref[...]
        var = jnp.mean(jnp.square(x), axis=-1, keepdims=True)
        x = x * lax.rsqrt(var + EPS)
        x = x * rscale_ref[...] * (HIDDEN ** -0.5)
        logits = jnp.dot(
            x.astype(jnp.bfloat16),
            rproj_ref[...].astype(jnp.bfloat16),
            preferred_element_type=jnp.float32,
        )
        probs = jax.nn.softmax(logits, axis=-1)

        iota = lax.broadcasted_iota(jnp.int32, probs.shape, 1)
        m1 = jnp.max(probs, axis=-1, keepdims=True)
        a1 = jnp.min(jnp.where(probs == m1, iota, NUM_EXPERTS), axis=-1,
                     keepdims=True)
        one1 = (iota == a1)
        probs2 = jnp.where(one1, -jnp.inf, probs)
        m2 = jnp.max(probs2, axis=-1, keepdims=True)
        a2 = jnp.min(jnp.where(probs2 == m2, iota, NUM_EXPERTS), axis=-1,
                     keepdims=True)
        one2 = (iota == a2)

        denom = m1 + m2 + 1e-20
        comb = (m1 * one1 + m2 * one2) / denom
        comb_ref[...] = comb * pes_ref[...]

    x = xbf_ref[...]
    g = jnp.dot(x, w1_ref[0].astype(jnp.bfloat16),
                preferred_element_type=jnp.float32)
    u = jnp.dot(x, w3_ref[0].astype(jnp.bfloat16),
                preferred_element_type=jnp.float32)
    h = jax.nn.gelu(g) * u
    y = jnp.dot(h.astype(jnp.bfloat16), w2_ref[0].astype(jnp.bfloat16),
                preferred_element_type=jnp.float32)
    lane = lax.broadcasted_iota(jnp.int32, (1, NUM_EXPERTS), 1)
    c = jnp.sum(comb_ref[...] * (lane == e).astype(jnp.float32), axis=-1,
                keepdims=True)
    contrib = c * y

    @pl.when(e == 0)
    def _():
        out_ref[...] = contrib

    @pl.when(e != 0)
    def _():
        out_ref[...] += contrib


@jax.jit
def kernel(hidden_states, router_input, router_scale, router_proj,
           per_expert_scale, w1, w2, w3):
    T, H = hidden_states.shape
    E = NUM_EXPERTS

    out = pl.pallas_call(
        _moe_body,
        grid=(E,),
        out_shape=jax.ShapeDtypeStruct((T, H), jnp.float32),
        in_specs=[
            pl.BlockSpec((T, H), lambda e: (0, 0)),
            pl.BlockSpec((T, H), lambda e: (0, 0)),
            pl.BlockSpec((1, H), lambda e: (0, 0)),
            pl.BlockSpec((H, E), lambda e: (0, 0)),
            pl.BlockSpec((1, E), lambda e: (0, 0)),
            pl.BlockSpec((1, H, DFF), lambda e: (e, 0, 0)),
            pl.BlockSpec((1, H, DFF), lambda e: (e, 0, 0)),
            pl.BlockSpec((1, DFF, H), lambda e: (e, 0, 0)),
        ],
        out_specs=pl.BlockSpec((T, H), lambda e: (0, 0)),
        scratch_shapes=[
            pltpu.VMEM((T, E), jnp.float32),
            pltpu.VMEM((T, H), jnp.bfloat16),
        ],
        compiler_params=pltpu.CompilerParams(
            dimension_semantics=("arbitrary",),
        ),
    )(hidden_states, router_input, router_scale.reshape(1, H), router_proj,
      per_expert_scale.reshape(1, E), w1, w3, w2)
    return out


# speedup vs baseline: 1.0313x; 1.0313x over previous
import jax
import jax.numpy as jnp
from jax import lax
from jax.experimental import pallas as pl
from jax.experimental.pallas import tpu as pltpu

HIDDEN = 768
NUM_EXPERTS = 8
TOP_K = 2
DFF = 1024
TOKENS = 2048
EPS = 1e-06


def _moe_body(hid_ref, rin_ref, rscale_ref, rproj_ref, pes_ref,
              w1_ref, w3_ref, w2_ref, out_ref, comb_ref, xbf_ref):
    e = pl.program_id(0)

    @pl.when(e == 0)
    def _():
        xbf_ref[...] = hid_ref[...].astype(jnp.bfloat16)
        x = rin_ref[...]
        var = jnp.mean(jnp.square(x), axis=-1, keepdims=True)
        x = x * lax.rsqrt(var + EPS)
        x = x * rscale_ref[...] * (HIDDEN ** -0.5)
        logits = jnp.dot(
            x.astype(jnp.bfloat16),
            rproj_ref[...].astype(jnp.bfloat16),
            preferred_element_type=jnp.float32,
        )
        probs = jax.nn.softmax(logits, axis=-1)

        iota = lax.broadcasted_iota(jnp.int32, probs.shape, 1)
        m1 = jnp.max(probs, axis=-1, keepdims=True)
        a1 = jnp.min(jnp.where(probs == m1, iota, NUM_EXPERTS), axis=-1,
                     keepdims=True)
        one1 = (iota == a1)
        probs2 = jnp.where(one1, -jnp.inf, probs)
        m2 = jnp.max(probs2, axis=-1, keepdims=True)
        a2 = jnp.min(jnp.where(probs2 == m2, iota, NUM_EXPERTS), axis=-1,
                     keepdims=True)
        one2 = (iota == a2)

        denom = m1 + m2 + 1e-20
        comb = (m1 * one1 + m2 * one2) / denom
        comb_ref[...] = comb * pes_ref[...]

    x = xbf_ref[...]
    g = jnp.dot(x, w1_ref[0].astype(jnp.bfloat16),
                preferred_element_type=jnp.float32).astype(jnp.bfloat16)
    u = jnp.dot(x, w3_ref[0].astype(jnp.bfloat16),
                preferred_element_type=jnp.float32).astype(jnp.bfloat16)
    h = jax.nn.gelu(g) * u
    y = jnp.dot(h, w2_ref[0].astype(jnp.bfloat16),
                preferred_element_type=jnp.float32)
    lane = lax.broadcasted_iota(jnp.int32, (1, NUM_EXPERTS), 1)
    c = jnp.sum(comb_ref[...] * (lane == e).astype(jnp.float32), axis=-1,
                keepdims=True)
    contrib = c * y

    @pl.when(e == 0)
    def _():
        out_ref[...] = contrib

    @pl.when(e != 0)
    def _():
        out_ref[...] += contrib


@jax.jit
def kernel(hidden_states, router_input, router_scale, router_proj,
           per_expert_scale, w1, w2, w3):
    T, H = hidden_states.shape
    E = NUM_EXPERTS

    out = pl.pallas_call(
        _moe_body,
        grid=(E,),
        out_shape=jax.ShapeDtypeStruct((T, H), jnp.float32),
        in_specs=[
            pl.BlockSpec((T, H), lambda e: (0, 0)),
            pl.BlockSpec((T, H), lambda e: (0, 0)),
            pl.BlockSpec((1, H), lambda e: (0, 0)),
            pl.BlockSpec((H, E), lambda e: (0, 0)),
            pl.BlockSpec((1, E), lambda e: (0, 0)),
            pl.BlockSpec((1, H, DFF), lambda e: (e, 0, 0)),
            pl.BlockSpec((1, H, DFF), lambda e: (e, 0, 0)),
            pl.BlockSpec((1, DFF, H), lambda e: (e, 0, 0)),
        ],
        out_specs=pl.BlockSpec((T, H), lambda e: (0, 0)),
        scratch_shapes=[
            pltpu.VMEM((T, E), jnp.float32),
            pltpu.VMEM((T, H), jnp.bfloat16),
        ],
        compiler_params=pltpu.CompilerParams(
            dimension_semantics=("arbitrary",),
        ),
    )(hidden_states, router_input, router_scale.reshape(1, H), router_proj,
      per_expert_scale.reshape(1, E), w1, w3, w2)
    return out
